# Initial kernel scaffold; baseline (speedup 1.0000x reference)
import jax
import jax.numpy as jnp
from jax.experimental import pallas as pl


def kernel(x, w1, b1, w2, b2):
    raise NotImplementedError("write your pallas kernel here")


# speedup vs baseline: 1.0624x; 1.0624x over previous
import functools
import math

import jax
import jax.numpy as jnp
from jax import lax
from jax.experimental import pallas as pl
from jax.experimental.pallas import tpu as pltpu

_INV_SQRT_2 = 0.7071067811865475


def _round_up(x, m):
    return -(-x // m) * m


def _ffn_kernel(x_ref, w1_ref, b1_ref, w2_ref, b2_ref, o_ref):
    j = pl.program_id(1)
    h = jnp.dot(x_ref[...], w1_ref[...].astype(jnp.bfloat16),
                preferred_element_type=jnp.float32)
    h = h + b1_ref[...]
    h = 0.5 * h * (1.0 + lax.erf(h * _INV_SQRT_2))
    y = jnp.dot(h.astype(jnp.bfloat16), w2_ref[...].astype(jnp.bfloat16),
                preferred_element_type=jnp.float32)

    @pl.when(j == 0)
    def _():
        o_ref[...] = y + b2_ref[...]

    @pl.when(j != 0)
    def _():
        o_ref[...] += y


@functools.partial(jax.jit, static_argnames=("tm", "th"))
def _ffn(x, w1, b1, w2, b2, *, tm=1024, th=512):
    orig_shape = x.shape
    D = orig_shape[-1]
    H = w1.shape[1]
    M = math.prod(orig_shape[:-1])

    tm = max(8, min(tm, _round_up(M, 8)))
    Mp = _round_up(M, tm)
    xb = x.reshape(M, D).astype(jnp.bfloat16)
    if Mp != M:
        xb = jnp.pad(xb, ((0, Mp - M), (0, 0)))

    while H % th:
        th //= 2
    Hp = H

    grid = (Mp // tm, Hp // th)
    b1_2d = b1.reshape(1, H)
    b2_2d = b2.reshape(1, D)

    x_item = 2
    w_item = jnp.dtype(w1.dtype).itemsize
    cost = pl.CostEstimate(
        flops=4 * Mp * D * H,
        transcendentals=Mp * H,
        bytes_accessed=(Mp * D * (x_item + 4)
                        + grid[0] * (2 * D * H + H) * w_item + D * w_item),
    )

    out2d = pl.pallas_call(
        _ffn_kernel,
        out_shape=jax.ShapeDtypeStruct((Mp, D), jnp.float32),
        grid=grid,
        in_specs=[
            pl.BlockSpec((tm, D), lambda i, j: (i, 0)),
            pl.BlockSpec((D, th), lambda i, j: (0, j)),
            pl.BlockSpec((1, th), lambda i, j: (0, j)),
            pl.BlockSpec((th, D), lambda i, j: (j, 0)),
            pl.BlockSpec((1, D), lambda i, j: (0, 0)),
        ],
        out_specs=pl.BlockSpec((tm, D), lambda i, j: (i, 0)),
        compiler_params=pltpu.CompilerParams(
            dimension_semantics=("parallel", "arbitrary"),
            vmem_limit_bytes=60 * 1024 * 1024,
        ),
        cost_estimate=cost,
    )(xb, w1, b1_2d, w2, b2_2d)

    if Mp != M:
        out2d = out2d[:M]
    return out2d.reshape(orig_shape)


def kernel(x, w1, b1, w2, b2):
    return _ffn(x, w1, b1, w2, b2)


# speedup vs baseline: 1.1418x; 1.0747x over previous
import functools
import math

import jax
import jax.numpy as jnp
from jax import lax
from jax.experimental import pallas as pl
from jax.experimental.pallas import tpu as pltpu

_INV_SQRT_2 = 0.7071067811865475


def _round_up(x, m):
    return -(-x // m) * m


def _ffn_kernel(x_ref, w1_ref, b1_ref, w2_ref, b2_ref, o_ref, acc_ref, sem):
    i = pl.program_id(0)
    j = pl.program_id(1)
    ni = pl.num_programs(0)
    nj = pl.num_programs(1)
    tm = acc_ref.shape[0]

    @pl.when((j == 0) & (i > 0))
    def _():
        pltpu.make_async_copy(
            acc_ref, o_ref.at[pl.ds((i - 1) * tm, tm), :], sem).wait()

    h = jnp.dot(x_ref[...], w1_ref[...].astype(jnp.bfloat16),
                preferred_element_type=jnp.float32)
    h = h + b1_ref[...]
    h = 0.5 * h * (1.0 + lax.erf(h * _INV_SQRT_2))
    y = jnp.dot(h, w2_ref[...], preferred_element_type=jnp.float32)

    @pl.when(j == 0)
    def _():
        acc_ref[...] = y + b2_ref[...]

    @pl.when(j != 0)
    def _():
        acc_ref[...] += y

    @pl.when(j == nj - 1)
    def _():
        pltpu.make_async_copy(
            acc_ref, o_ref.at[pl.ds(i * tm, tm), :], sem).start()

    @pl.when((j == nj - 1) & (i == ni - 1))
    def _():
        pltpu.make_async_copy(
            acc_ref, o_ref.at[pl.ds(i * tm, tm), :], sem).wait()


@functools.partial(jax.jit, static_argnames=("tm", "th"))
def _ffn(x, w1, b1, w2, b2, *, tm=1024, th=1024):
    orig_shape = x.shape
    D = orig_shape[-1]
    H = w1.shape[1]
    M = math.prod(orig_shape[:-1])

    tm = max(8, min(tm, _round_up(M, 8)))
    Mp = _round_up(M, tm)
    xb = x.reshape(M, D).astype(jnp.bfloat16)
    if Mp != M:
        xb = jnp.pad(xb, ((0, Mp - M), (0, 0)))

    while H % th:
        th //= 2

    grid = (Mp // tm, H // th)
    b1_2d = b1.reshape(1, H)
    b2_2d = b2.reshape(1, D)

    w_item = jnp.dtype(w1.dtype).itemsize
    cost = pl.CostEstimate(
        flops=4 * Mp * D * H,
        transcendentals=Mp * H,
        bytes_accessed=(Mp * D * (2 + 4)
                        + grid[0] * (2 * D * H + H) * w_item + D * w_item),
    )

    out2d = pl.pallas_call(
        _ffn_kernel,
        out_shape=jax.ShapeDtypeStruct((Mp, D), jnp.float32),
        grid=grid,
        in_specs=[
            pl.BlockSpec((tm, D), lambda i, j: (i, 0)),
            pl.BlockSpec((D, th), lambda i, j: (0, j)),
            pl.BlockSpec((1, th), lambda i, j: (0, j)),
            pl.BlockSpec((th, D), lambda i, j: (j, 0)),
            pl.BlockSpec((1, D), lambda i, j: (0, 0)),
        ],
        out_specs=pl.BlockSpec(memory_space=pl.ANY),
        scratch_shapes=[
            pltpu.VMEM((tm, D), jnp.float32),
            pltpu.SemaphoreType.DMA,
        ],
        compiler_params=pltpu.CompilerParams(
            dimension_semantics=("arbitrary", "arbitrary"),
            vmem_limit_bytes=63 * 1024 * 1024,
        ),
        cost_estimate=cost,
    )(xb, w1, b1_2d, w2, b2_2d)

    if Mp != M:
        out2d = out2d[:M]
    return out2d.reshape(orig_shape)


def kernel(x, w1, b1, w2, b2):
    return _ffn(x, w1, b1, w2, b2)
